# baseline (speedup 1.0000x reference)
import functools

import jax
import jax.numpy as jnp
from jax import lax
from jax.experimental import pallas as pl
from jax.experimental.pallas import tpu as pltpu
from jax.experimental.pallas import tpu_sc as plsc

HIDDEN_DIM = 768
EXPERT_NUM = 64
TOP_K = 8
N_TOKENS = 32768

TILE = 4096

SC_CORES = 2
SC_SUBCORES = 16
SC_WORKERS = SC_CORES * SC_SUBCORES
TOK_PER_W = N_TOKENS // SC_WORKERS
SUB_C = 128
N_SUB = TOK_PER_W // SUB_C
LANES = 16


def _router_kernel(x_ref, w_ref, b_ref, router_ref, weight_ref, idx_ref,
                   idxt_ref):
    r = lax.dot_general(
        x_ref[...], w_ref[...],
        dimension_numbers=(((1,), (1,)), ((), ())),
        preferred_element_type=jnp.float32) + b_ref[...]
    router_ref[...] = r

    rt = r.T
    e_iota = lax.broadcasted_iota(jnp.int32, (EXPERT_NUM, TILE),
                                  0).astype(jnp.float32)

    vals = rt
    top_vals = []
    top_idx = []
    for _ in range(TOP_K):
        m = jnp.max(vals, axis=0, keepdims=True)
        is_max = vals == m
        idx = jnp.min(jnp.where(is_max, e_iota, float(EXPERT_NUM)), axis=0,
                      keepdims=True)
        top_vals.append(m)
        top_idx.append(idx)
        vals = jnp.where(e_iota == idx, -jnp.inf, vals)

    vals8t = jnp.concatenate(top_vals, axis=0)
    idx8t = jnp.concatenate(top_idx, axis=0).astype(jnp.int32)
    idx_ref[...] = idx8t.T
    idxt_ref[...] = idx8t

    e = jnp.exp(vals8t - vals8t[0:1, :])
    weight_ref[...] = (e / jnp.sum(e, axis=0, keepdims=True)).T


def _mask_sc_body(idxt_hbm, zeros_hbm, mask_hbm, idx_v, buf):
    c = lax.axis_index("c")
    s = lax.axis_index("s")
    wid = s * SC_CORES + c
    wbase = wid * TOK_PER_W

    pltpu.sync_copy(idxt_hbm.at[:, pl.ds(wbase, TOK_PER_W)], idx_v)
    pltpu.sync_copy(zeros_hbm, buf)

    l_iota = lax.iota(jnp.int32, LANES)
    ones = jnp.ones((LANES,), jnp.int32)
    zeros = jnp.zeros((LANES,), jnp.int32)

    for sub in range(N_SUB):
        coff = sub * SUB_C
        for k in range(TOP_K):
            k_vec = jnp.full((LANES,), k, jnp.int32)
            for g in range(SUB_C // LANES):
                n_vec = g * LANES + l_iota
                e_vec = idx_v[k, pl.ds(coff + g * LANES, LANES)]
                plsc.store_scatter(buf, [e_vec, k_vec, n_vec], ones)
        pltpu.sync_copy(buf, mask_hbm.at[:, :, pl.ds(wbase + coff, SUB_C)])
        for k in range(TOP_K):
            k_vec = jnp.full((LANES,), k, jnp.int32)
            for g in range(SUB_C // LANES):
                n_vec = g * LANES + l_iota
                e_vec = idx_v[k, pl.ds(coff + g * LANES, LANES)]
                plsc.store_scatter(buf, [e_vec, k_vec, n_vec], zeros)


_mask_sc = functools.partial(
    pl.kernel,
    mesh=plsc.VectorSubcoreMesh(core_axis_name="c", subcore_axis_name="s"),
    out_type=jax.ShapeDtypeStruct((EXPERT_NUM, TOP_K, N_TOKENS), jnp.int32),
    scratch_types=[
        pltpu.VMEM((TOP_K, TOK_PER_W), jnp.int32),
        pltpu.VMEM((EXPERT_NUM, TOP_K, SUB_C), jnp.int32),
    ],
    compiler_params=pltpu.CompilerParams(needs_layout_passes=False),
)(_mask_sc_body)


@jax.jit
def kernel(x, gate_w, gate_b):
    b2 = gate_b.reshape(1, EXPERT_NUM)
    grid = (N_TOKENS // TILE,)

    out_shapes = (
        jax.ShapeDtypeStruct((N_TOKENS, EXPERT_NUM), jnp.float32),
        jax.ShapeDtypeStruct((N_TOKENS, TOP_K), jnp.float32),
        jax.ShapeDtypeStruct((N_TOKENS, TOP_K), jnp.int32),
        jax.ShapeDtypeStruct((TOP_K, N_TOKENS), jnp.int32),
    )
    in_specs = [
        pl.BlockSpec((TILE, HIDDEN_DIM), lambda i: (i, 0)),
        pl.BlockSpec((EXPERT_NUM, HIDDEN_DIM), lambda i: (0, 0)),
        pl.BlockSpec((1, EXPERT_NUM), lambda i: (0, 0)),
    ]
    out_specs = (
        pl.BlockSpec((TILE, EXPERT_NUM), lambda i: (i, 0)),
        pl.BlockSpec((TILE, TOP_K), lambda i: (i, 0)),
        pl.BlockSpec((TILE, TOP_K), lambda i: (i, 0)),
        pl.BlockSpec((TOP_K, TILE), lambda i: (0, i)),
    )
    router, weight, idx, idxt = pl.pallas_call(
        _router_kernel,
        grid=grid,
        in_specs=in_specs,
        out_specs=out_specs,
        out_shape=out_shapes,
        compiler_params=pltpu.CompilerParams(
            dimension_semantics=("parallel",),
        ),
    )(x, gate_w, b2)

    zeros_tile = jnp.zeros((EXPERT_NUM, TOP_K, SUB_C), jnp.int32)
    mask = _mask_sc(idxt, zeros_tile)
    return (router, weight, idx, mask)
